# Initial kernel scaffold; baseline (speedup 1.0000x reference)
import jax
import jax.numpy as jnp
from jax.experimental import pallas as pl


def kernel(x, labels, centre):
    raise NotImplementedError("write your pallas kernel here")


# speedup vs baseline: 3.9132x; 3.9132x over previous
import functools

import jax
import jax.numpy as jnp
from jax import lax
from jax.experimental import pallas as pl
from jax.experimental.pallas import tpu as pltpu
from jax.experimental.pallas import tpu_sc as plsc

CLS = 1000
FEAT = 64
N = 16384
L = 16
NC = 2
NS = 16
NW = NC * NS
BPW = N // NW
HPW = N // NS
HROW = HPW // 128
HIST_PAD = 1024


def _body(x_hbm, lab_hbm, centre_hbm, out_hbm,
          x_v, centre_v, labf_v, labh_v, bins_v, ones_v, hist_v,
          inv_v, mm_v, mmst_v, acc_v, hist_sh, mm_sh,
          sem_x, sem_c, sem_l):
    c = lax.axis_index("c")
    s = lax.axis_index("s")
    wid = c * NS + s

    pltpu.sync_copy(lab_hbm.at[pl.ds(s * HPW, HPW)], labh_v)
    pending = [
        pltpu.async_copy(x_hbm.at[pl.ds(wid * BPW * FEAT, BPW * FEAT)], x_v, sem_x),
        pltpu.async_copy(centre_hbm, centre_v, sem_c),
        pltpu.async_copy(lab_hbm.at[pl.ds(wid * BPW, BPW)], labf_v, sem_l),
    ]

    zero16 = jnp.zeros((L,), jnp.float32)
    for j in range(HIST_PAD // L):
        hist_v[pl.ds(j * L, L)] = zero16

    @pl.when(s == 0)
    def _():
        pltpu.sync_copy(hist_v, hist_sh)

    minv = jnp.full((L,), 1e9, jnp.float32)
    maxv = jnp.full((L,), -1e9, jnp.float32)
    for j in range(HPW // L):
        v = labh_v[pl.ds(j * L, L)].astype(jnp.float32)
        minv = jnp.minimum(minv, v)
        maxv = jnp.maximum(maxv, v)
    mmst_v[pl.ds(0, L)] = minv
    mmst_v[pl.ds(L, L)] = maxv
    pltpu.sync_copy(mmst_v, mm_sh.at[pl.ds(2 * L * s, 2 * L)])
    plsc.subcore_barrier()

    pltpu.sync_copy(mm_sh, mm_v)
    for r in range(NS):
        minv = jnp.minimum(minv, mm_v[pl.ds(2 * L * r, L)])
        maxv = jnp.maximum(maxv, mm_v[pl.ds(2 * L * r + L, L)])
    vmin, vmax = minv[0], maxv[0]
    for j in range(1, L):
        vmin = jnp.minimum(vmin, minv[j])
        vmax = jnp.maximum(vmax, maxv[j])
    span = vmax - vmin
    span = jnp.where(span == 0.0, 1.0, span)

    ones16 = jnp.ones((L,), jnp.float32)
    for k in range(128 // L):
        ones_v[pl.ds(k * L, L)] = ones16
    for r in range(HROW):
        for k in range(128 // L):
            lab = labh_v[pl.ds((r * (128 // L) + k) * L, L)].astype(jnp.float32)
            t = (lab - vmin) / span * jnp.float32(CLS)
            b = jnp.clip(t.astype(jnp.int32), 0, CLS - 1)
            bins_v[r, pl.ds(k * L, L)] = b
    for r in range(HROW):
        pltpu.sync_copy(ones_v, hist_sh.at[bins_v.at[r]], add=True)
    plsc.subcore_barrier()

    pltpu.sync_copy(hist_sh, hist_v)
    labf_done = pending.pop()
    labf_done.wait()
    for j in range(BPW // L):
        lab = labf_v[pl.ds(j * L, L)]
        cnt = plsc.load_gather(hist_v, [lab])
        inv_v[pl.ds(j * L, L)] = 1.0 / cnt

    for h in pending:
        h.wait()

    def step(blk, acc):
        labc = labf_v[pl.ds(blk * L, L)]
        invc = inv_v[pl.ds(blk * L, L)]
        for j in range(L):
            xb = (blk * L + j) * FEAT
            cb = labc[j] * FEAT
            v = jnp.zeros((L,), jnp.float32)
            for k in range(FEAT // L):
                d = x_v[pl.ds(xb + k * L, L)] - centre_v[pl.ds(cb + k * L, L)]
                v = v + d * d
            acc = acc + v * invc[j]
        return acc

    acc = lax.fori_loop(0, BPW // L, step, jnp.zeros((L,), jnp.float32))
    acc_v[...] = acc
    pltpu.sync_copy(acc_v, out_hbm.at[pl.ds(wid * L, L)])


_sc_call = functools.partial(
    pl.kernel,
    mesh=plsc.VectorSubcoreMesh(core_axis_name="c", subcore_axis_name="s"),
    out_type=jax.ShapeDtypeStruct((NW * L,), jnp.float32),
    scratch_types=[
        pltpu.VMEM((BPW * FEAT,), jnp.float32),
        pltpu.VMEM((CLS * FEAT,), jnp.float32),
        pltpu.VMEM((BPW,), jnp.int32),
        pltpu.VMEM((HPW,), jnp.int32),
        pltpu.VMEM((HROW, 128), jnp.int32),
        pltpu.VMEM((128,), jnp.float32),
        pltpu.VMEM((HIST_PAD,), jnp.float32),
        pltpu.VMEM((BPW,), jnp.float32),
        pltpu.VMEM((2 * L * NS,), jnp.float32),
        pltpu.VMEM((2 * L,), jnp.float32),
        pltpu.VMEM((L,), jnp.float32),
        pltpu.VMEM_SHARED((HIST_PAD,), jnp.float32),
        pltpu.VMEM_SHARED((2 * L * NS,), jnp.float32),
        pltpu.SemaphoreType.DMA,
        pltpu.SemaphoreType.DMA,
        pltpu.SemaphoreType.DMA,
    ],
    compiler_params=pltpu.CompilerParams(needs_layout_passes=False),
)(_body)


def kernel(x, labels, centre):
    partial = _sc_call(x.reshape(-1), labels, centre.reshape(-1))
    return jnp.sqrt(jnp.sum(partial))
